# baseline (speedup 1.0000x reference)
import functools

import jax
import jax.numpy as jnp
from jax import lax
from jax.experimental import pallas as pl
from jax.experimental.pallas import tpu as pltpu
from jax.experimental.pallas import tpu_sc as plsc

NC = 2
NS = 16
CH = 80
RB = 400


def _bcast_lane(v16, j):
  idx = jnp.full((16, 1), j, dtype=jnp.int32)
  dnums = lax.GatherDimensionNumbers(
      offset_dims=(), collapsed_slice_dims=(0,), start_index_map=(0,))
  return lax.gather(v16, idx, dnums, slice_sizes=(1,),
                    mode=lax.GatherScatterMode.PROMISE_IN_BOUNDS)


def _make_spmm(n_rows, n_tab, D, EPW, col_split):
  NCHUNK = EPW // CH
  RPSF = (-(-n_rows // NS) + 7) // 8 * 8
  RPSL = n_rows - (NS - 1) * RPSF
  assert RPSL > 0
  G16 = CH // 16
  DV = D // 16
  mesh = plsc.VectorSubcoreMesh(core_axis_name="c", subcore_axis_name="s",
                                num_cores=NC, num_subcores=NS)

  NBUF = 4
  assert NCHUNK % NBUF == 0 and NCHUNK >= NBUF

  @functools.partial(
      pl.kernel,
      out_type=jax.ShapeDtypeStruct((2 * n_rows, D), jnp.float32),
      mesh=mesh,
      scratch_types=[
          pltpu.VMEM_SHARED((n_rows, D), jnp.float32),
      ]
      + [pltpu.VMEM((CH, D), jnp.float32) for _ in range(NBUF)]
      + [pltpu.VMEM((CH,), jnp.int32) for _ in range(NBUF)]
      + [pltpu.VMEM((CH,), jnp.int32) for _ in range(NBUF)]
      + [pltpu.VMEM((CH,), jnp.float32) for _ in range(NBUF)]
      + [pltpu.SemaphoreType.DMA for _ in range(3 * NBUF + 1)],
  )
  def spmm(tab_hbm, src_hbm, dst_hbm, w_hbm, zer_hbm, out_hbm,
           accum, *bufs):
    rows = bufs[:NBUF]
    srcc = bufs[NBUF:2 * NBUF]
    dstc = bufs[2 * NBUF:3 * NBUF]
    wc = bufs[3 * NBUF:4 * NBUF]
    esem = bufs[4 * NBUF:5 * NBUF]
    gsem = bufs[5 * NBUF:6 * NBUF]
    ssem = bufs[6 * NBUF:7 * NBUF]
    zsem = bufs[7 * NBUF]
    c = lax.axis_index("c")
    s = lax.axis_index("s")
    if col_split:
      ebase_src = (c * NS + s) * EPW
      ebase_dw = s * EPW
    else:
      sl = s * NC + c
      ebase_src = sl * EPW
      ebase_dw = sl * EPW
    ebase_src = pl.multiple_of(ebase_src, 8)
    ebase_dw = pl.multiple_of(ebase_dw, 8)
    rbase = pl.multiple_of(s * RPSF, 8)

    @pl.when(s < NS - 1)
    def _():
      pltpu.async_copy(zer_hbm, accum.at[pl.ds(rbase, RPSF)], zsem).wait()

    @pl.when(s == NS - 1)
    def _():
      pltpu.async_copy(zer_hbm.at[pl.ds(0, RPSL)],
                       accum.at[pl.ds(rbase, RPSL)], zsem).wait()

    plsc.subcore_barrier()

    def start_edges(g, b):
      gb = pl.multiple_of(g * CH, CH)
      pltpu.async_copy(src_hbm.at[pl.ds(ebase_src + gb, CH)], srcc[b], esem[b])
      pltpu.async_copy(dst_hbm.at[pl.ds(ebase_dw + gb, CH)], dstc[b], esem[b])
      pltpu.async_copy(w_hbm.at[pl.ds(ebase_dw + gb, CH)], wc[b], esem[b])

    def wait_edges(b):
      pltpu.make_async_copy(src_hbm.at[pl.ds(0, CH)], srcc[b], esem[b]).wait()
      pltpu.make_async_copy(dst_hbm.at[pl.ds(0, CH)], dstc[b], esem[b]).wait()
      pltpu.make_async_copy(w_hbm.at[pl.ds(0, CH)], wc[b], esem[b]).wait()

    def start_gather(b):
      pltpu.async_copy(tab_hbm.at[srcc[b]], rows[b], gsem[b])

    def wait_gather(b):
      pltpu.make_async_copy(tab_hbm.at[srcc[b]], rows[b], gsem[b]).wait()

    def wait_scatter(b):
      pltpu.make_async_copy(rows[b], accum.at[dstc[b]], ssem[b]).wait()

    start_edges(0, 0)
    start_edges(1, 1)
    wait_edges(0)
    start_gather(0)

    def outer(gg, carry):
      for b in range(NBUF):
        g = gg * NBUF + b
        nb2 = (b + 2) % NBUF
        nb1 = (b + 1) % NBUF
        @pl.when(jnp.logical_and(g >= 2, g + 2 < NCHUNK))
        def _():
          wait_scatter(nb2)

        @pl.when(g + 2 < NCHUNK)
        def _():
          start_edges(g + 2, nb2)

        @pl.when(g + 1 < NCHUNK)
        def _():
          wait_edges(nb1)
          start_gather(nb1)

        wait_gather(b)
        for grp in range(G16):
          w16 = wc[b][pl.ds(grp * 16, 16)]
          for j in range(16):
            wj = _bcast_lane(w16, j)
            e = grp * 16 + j
            for k in range(DV):
              csl = pl.ds(k * 16, 16)
              rows[b][e, csl] = rows[b][e, csl] * wj
        pltpu.async_copy(rows[b], accum.at[dstc[b]], ssem[b], add=True)
      return carry

    lax.fori_loop(0, NCHUNK // NBUF, outer, 0)
    for b in range(NBUF):
      wait_scatter(b)
    plsc.subcore_barrier()

    obase = pl.multiple_of(c * n_rows + rbase, 8)

    @pl.when(s < NS - 1)
    def _():
      pltpu.sync_copy(accum.at[pl.ds(rbase, RPSF)],
                      out_hbm.at[pl.ds(obase, RPSF)])

    @pl.when(s == NS - 1)
    def _():
      pltpu.sync_copy(accum.at[pl.ds(rbase, RPSL)],
                      out_hbm.at[pl.ds(obase, RPSL)])

  return spmm


def _mm1(x, W1, n):
  nb = n // RB

  def body(x_ref, w_ref, o_ref):
    o_ref[...] = jnp.dot(x_ref[...], w_ref[...],
                         preferred_element_type=jnp.float32)

  return pl.pallas_call(
      body,
      grid=(NC, nb),
      in_specs=[
          pl.BlockSpec((RB, x.shape[1]), lambda c, i: (i, 0)),
          pl.BlockSpec((W1.shape[0], 128), lambda c, i: (0, c)),
      ],
      out_specs=pl.BlockSpec((RB, 128), lambda c, i, _nb=nb: (c * _nb + i, 0)),
      out_shape=jax.ShapeDtypeStruct((2 * n, 128), jnp.float32),
  )(x, W1)


def _mm2(h1, W2p, n):
  nb = n // RB

  def body(a_ref, b_ref, w_ref, o_ref):
    w = w_ref[...]
    a = jnp.maximum(a_ref[...], 0.0)
    b = jnp.maximum(b_ref[...], 0.0)
    o_ref[...] = (
        jnp.dot(a, w[:128], preferred_element_type=jnp.float32)
        + jnp.dot(b, w[128:], preferred_element_type=jnp.float32))

  return pl.pallas_call(
      body,
      grid=(nb,),
      in_specs=[
          pl.BlockSpec((RB, 128), lambda i: (i, 0)),
          pl.BlockSpec((RB, 128), lambda i, _nb=nb: (_nb + i, 0)),
          pl.BlockSpec(W2p.shape, lambda i: (0, 0)),
      ],
      out_specs=pl.BlockSpec((RB, 128), lambda i: (i, 0)),
      out_shape=jax.ShapeDtypeStruct((n, 128), jnp.float32),
  )(h1, h1, W2p)


def _finish(p, n, ncls):
  nb = n // RB

  def body(a_ref, b_ref, o_ref):
    z = jnp.maximum(a_ref[:, :ncls] + b_ref[:, :ncls], 0.0)
    z = z - jnp.max(z, axis=1, keepdims=True)
    o_ref[...] = z - jnp.log(jnp.sum(jnp.exp(z), axis=1, keepdims=True))

  return pl.pallas_call(
      body,
      grid=(nb,),
      in_specs=[
          pl.BlockSpec((RB, 128), lambda i: (i, 0)),
          pl.BlockSpec((RB, 128), lambda i, _nb=nb: (_nb + i, 0)),
      ],
      out_specs=pl.BlockSpec((RB, ncls), lambda i: (i, 0)),
      out_shape=jax.ShapeDtypeStruct((n, ncls), jnp.float32),
  )(p, p)


@jax.jit
def kernel(x, edge_index, edge_weight, W1, W2):
  n = x.shape[0]
  ncls = W2.shape[1]
  e = edge_weight.shape[0]

  quant = NC * NS * CH * 4
  ep = ((e + quant - 1) // quant) * quant
  pad = ep - e
  src = jnp.concatenate([edge_index[1], jnp.zeros((pad,), jnp.int32)])
  dst = jnp.concatenate([edge_index[0], jnp.zeros((pad,), jnp.int32)])
  w = jnp.concatenate([edge_weight, jnp.zeros((pad,), jnp.float32)])
  src2 = jnp.concatenate([src, src + n])

  rpsf = (-(-n // NS) + 7) // 8 * 8
  zer = jnp.zeros((rpsf, 128), jnp.float32)
  W2p = jnp.pad(W2, ((0, 0), (0, 128 - ncls)))

  sup1 = _mm1(x, W1, n)
  spmm1 = _make_spmm(n, 2 * n, 128, ep // NS, col_split=True)
  h1 = spmm1(sup1, src2, dst, w, zer)
  sup2 = _mm2(h1, W2p, n)
  spmm2 = _make_spmm(n, n, 128, ep // (NC * NS), col_split=False)
  p = spmm2(sup2, src, dst, w, zer)
  return _finish(p, n, ncls)


# speedup vs baseline: 1.0146x; 1.0146x over previous
import functools

import jax
import jax.numpy as jnp
from jax import lax
from jax.experimental import pallas as pl
from jax.experimental.pallas import tpu as pltpu
from jax.experimental.pallas import tpu_sc as plsc

NC = 2
NS = 16
CH = 32
NBUF = 8
ED_AHEAD = 6
G_AHEAD = 4
RB = 400


def _bcast_lane(v16, j):
  idx = jnp.full((16, 1), j, dtype=jnp.int32)
  dnums = lax.GatherDimensionNumbers(
      offset_dims=(), collapsed_slice_dims=(0,), start_index_map=(0,))
  return lax.gather(v16, idx, dnums, slice_sizes=(1,),
                    mode=lax.GatherScatterMode.PROMISE_IN_BOUNDS)


def _make_spmm(n_rows, n_tab, D, EPW, col_split, tab_in_spmem=False):
  NCHUNK = EPW // CH
  RPSF = (-(-n_rows // NS) + 7) // 8 * 8
  RPSL = n_rows - (NS - 1) * RPSF
  assert RPSL > 0
  G16 = CH // 16
  DV = D // 16
  mesh = plsc.VectorSubcoreMesh(core_axis_name="c", subcore_axis_name="s",
                                num_cores=NC, num_subcores=NS)

  assert NCHUNK % NBUF == 0 and NCHUNK >= NBUF + ED_AHEAD

  @functools.partial(
      pl.kernel,
      out_type=jax.ShapeDtypeStruct((2 * n_rows, D), jnp.float32),
      mesh=mesh,
      scratch_types=[
          pltpu.VMEM_SHARED((n_rows, D), jnp.float32),
      ]
      + ([pltpu.VMEM_SHARED((n_tab, D), jnp.float32)] if tab_in_spmem else [])
      + [pltpu.VMEM((CH, D), jnp.float32) for _ in range(NBUF)]
      + [pltpu.VMEM((CH,), jnp.int32) for _ in range(NBUF)]
      + [pltpu.VMEM((CH,), jnp.int32) for _ in range(NBUF)]
      + [pltpu.VMEM((CH,), jnp.float32) for _ in range(NBUF)]
      + [pltpu.SemaphoreType.DMA for _ in range(3 * NBUF + 2)],
  )
  def spmm(tab_hbm, src_hbm, dst_hbm, w_hbm, zer_hbm, out_hbm,
           accum, *bufs):
    if tab_in_spmem:
      tabs = bufs[0]
      bufs = bufs[1:]
    rows = bufs[:NBUF]
    srcc = bufs[NBUF:2 * NBUF]
    dstc = bufs[2 * NBUF:3 * NBUF]
    wc = bufs[3 * NBUF:4 * NBUF]
    esem = bufs[4 * NBUF:5 * NBUF]
    gsem = bufs[5 * NBUF:6 * NBUF]
    ssem = bufs[6 * NBUF:7 * NBUF]
    zsem = bufs[7 * NBUF]
    tsem = bufs[7 * NBUF + 1]
    c = lax.axis_index("c")
    s = lax.axis_index("s")
    if col_split:
      ebase_src = (c * NS + s) * EPW
      ebase_dw = s * EPW
    else:
      sl = s * NC + c
      ebase_src = sl * EPW
      ebase_dw = sl * EPW
    ebase_src = pl.multiple_of(ebase_src, 8)
    ebase_dw = pl.multiple_of(ebase_dw, 8)
    rbase = pl.multiple_of(s * RPSF, 8)

    @pl.when(s < NS - 1)
    def _():
      pltpu.async_copy(zer_hbm, accum.at[pl.ds(rbase, RPSF)], zsem).wait()

    @pl.when(s == NS - 1)
    def _():
      pltpu.async_copy(zer_hbm.at[pl.ds(0, RPSL)],
                       accum.at[pl.ds(rbase, RPSL)], zsem).wait()

    if tab_in_spmem:
      TPSF = (-(-n_tab // NS) + 7) // 8 * 8
      TPSL = n_tab - (NS - 1) * TPSF
      assert TPSL > 0
      tbase = pl.multiple_of(s * TPSF, 8)

      @pl.when(s < NS - 1)
      def _():
        pltpu.async_copy(tab_hbm.at[pl.ds(tbase, TPSF)],
                         tabs.at[pl.ds(tbase, TPSF)], tsem).wait()

      @pl.when(s == NS - 1)
      def _():
        pltpu.async_copy(tab_hbm.at[pl.ds(tbase, TPSL)],
                         tabs.at[pl.ds(tbase, TPSL)], tsem).wait()

    plsc.subcore_barrier()

    def start_edges(g, b):
      gb = pl.multiple_of(g * CH, CH)
      pltpu.async_copy(src_hbm.at[pl.ds(ebase_src + gb, CH)], srcc[b], esem[b])
      pltpu.async_copy(dst_hbm.at[pl.ds(ebase_dw + gb, CH)], dstc[b], esem[b])
      pltpu.async_copy(w_hbm.at[pl.ds(ebase_dw + gb, CH)], wc[b], esem[b])

    def wait_edges(b):
      pltpu.make_async_copy(src_hbm.at[pl.ds(0, CH)], srcc[b], esem[b]).wait()
      pltpu.make_async_copy(dst_hbm.at[pl.ds(0, CH)], dstc[b], esem[b]).wait()
      pltpu.make_async_copy(w_hbm.at[pl.ds(0, CH)], wc[b], esem[b]).wait()

    tab_ref = tabs if tab_in_spmem else tab_hbm

    def start_gather(b):
      pltpu.async_copy(tab_ref.at[srcc[b]], rows[b], gsem[b])

    def wait_gather(b):
      pltpu.make_async_copy(tab_ref.at[srcc[b]], rows[b], gsem[b]).wait()

    def wait_scatter(b):
      pltpu.make_async_copy(rows[b], accum.at[dstc[b]], ssem[b]).wait()

    for k in range(ED_AHEAD):
      start_edges(k, k)
    for k in range(G_AHEAD):
      wait_edges(k)
      start_gather(k)

    def outer(gg, carry):
      for b in range(NBUF):
        g = gg * NBUF + b
        se = (b + ED_AHEAD) % NBUF
        sg = (b + G_AHEAD) % NBUF
        @pl.when(jnp.logical_and(g >= NBUF - ED_AHEAD, g + ED_AHEAD < NCHUNK))
        def _():
          wait_scatter(se)

        @pl.when(g + ED_AHEAD < NCHUNK)
        def _():
          start_edges(g + ED_AHEAD, se)

        @pl.when(g + G_AHEAD < NCHUNK)
        def _():
          wait_edges(sg)
          start_gather(sg)

        wait_gather(b)
        for grp in range(G16):
          w16 = wc[b][pl.ds(grp * 16, 16)]
          for j in range(16):
            wj = _bcast_lane(w16, j)
            e = grp * 16 + j
            for k in range(DV):
              csl = pl.ds(k * 16, 16)
              rows[b][e, csl] = rows[b][e, csl] * wj
        pltpu.async_copy(rows[b], accum.at[dstc[b]], ssem[b], add=True)
      return carry

    lax.fori_loop(0, NCHUNK // NBUF, outer, 0)
    for b in range(NBUF):
      wait_scatter(b)
    plsc.subcore_barrier()

    obase = pl.multiple_of(c * n_rows + rbase, 8)

    @pl.when(s < NS - 1)
    def _():
      pltpu.sync_copy(accum.at[pl.ds(rbase, RPSF)],
                      out_hbm.at[pl.ds(obase, RPSF)])

    @pl.when(s == NS - 1)
    def _():
      pltpu.sync_copy(accum.at[pl.ds(rbase, RPSL)],
                      out_hbm.at[pl.ds(obase, RPSL)])

  return spmm


def _mm1(x, W1, n):
  nb = n // RB

  def body(x_ref, w_ref, o_ref):
    o_ref[...] = jnp.dot(x_ref[...], w_ref[...],
                         preferred_element_type=jnp.float32)

  return pl.pallas_call(
      body,
      grid=(NC, nb),
      in_specs=[
          pl.BlockSpec((RB, x.shape[1]), lambda c, i: (i, 0)),
          pl.BlockSpec((W1.shape[0], 128), lambda c, i: (0, c)),
      ],
      out_specs=pl.BlockSpec((RB, 128), lambda c, i, _nb=nb: (c * _nb + i, 0)),
      out_shape=jax.ShapeDtypeStruct((2 * n, 128), jnp.float32),
  )(x, W1)


def _mm2(h1, W2, n):
  nb = n // RB
  ncls = W2.shape[1]

  def body(a_ref, b_ref, w_ref, o_ref):
    w = w_ref[...]
    a = jnp.maximum(a_ref[...], 0.0)
    b = jnp.maximum(b_ref[...], 0.0)
    o_ref[...] = (
        jnp.dot(a, w[:128], preferred_element_type=jnp.float32)
        + jnp.dot(b, w[128:], preferred_element_type=jnp.float32))

  return pl.pallas_call(
      body,
      grid=(nb,),
      in_specs=[
          pl.BlockSpec((RB, 128), lambda i: (i, 0)),
          pl.BlockSpec((RB, 128), lambda i, _nb=nb: (_nb + i, 0)),
          pl.BlockSpec(W2.shape, lambda i: (0, 0)),
      ],
      out_specs=pl.BlockSpec((RB, W2.shape[1]), lambda i: (i, 0)),
      out_shape=jax.ShapeDtypeStruct((n, W2.shape[1]), jnp.float32),
  )(h1, h1, W2)


def _finish(p, n, ncls):
  nb = n // RB

  def body(a_ref, b_ref, o_ref):
    z = jnp.maximum(a_ref[:, :ncls] + b_ref[:, :ncls], 0.0)
    z = z - jnp.max(z, axis=1, keepdims=True)
    o_ref[...] = z - jnp.log(jnp.sum(jnp.exp(z), axis=1, keepdims=True))

  return pl.pallas_call(
      body,
      grid=(nb,),
      in_specs=[
          pl.BlockSpec((RB, 128), lambda i: (i, 0)),
          pl.BlockSpec((RB, 128), lambda i, _nb=nb: (_nb + i, 0)),
      ],
      out_specs=pl.BlockSpec((RB, ncls), lambda i: (i, 0)),
      out_shape=jax.ShapeDtypeStruct((n, ncls), jnp.float32),
  )(p, p)


@jax.jit
def kernel(x, edge_index, edge_weight, W1, W2):
  n = x.shape[0]
  ncls = W2.shape[1]
  e = edge_weight.shape[0]

  quant = NC * NS * CH * NBUF
  ep = ((e + quant - 1) // quant) * quant
  pad = ep - e
  src = jnp.concatenate([edge_index[1], jnp.zeros((pad,), jnp.int32)])
  dst = jnp.concatenate([edge_index[0], jnp.zeros((pad,), jnp.int32)])
  w = jnp.concatenate([edge_weight, jnp.zeros((pad,), jnp.float32)])
  src2 = jnp.concatenate([src, src + n])

  rpsf = (-(-n // NS) + 7) // 8 * 8
  zer = jnp.zeros((rpsf, 128), jnp.float32)
  W2p = jnp.pad(W2, ((0, 0), (0, 128 - ncls)))

  sup1 = _mm1(x, W1, n)
  spmm1 = _make_spmm(n, 2 * n, 128, ep // NS, col_split=True)
  h1 = spmm1(sup1, src2, dst, w, zer)
  sup2 = _mm2(h1, W2p, n)
  spmm2 = _make_spmm(n, n, 128, ep // (NC * NS), col_split=False)
  p = spmm2(sup2, src, dst, w, zer)
  return _finish(p, n, ncls)
